# baseline (speedup 1.0000x reference)
import functools

import jax
import jax.numpy as jnp
from jax import lax
from jax.experimental import pallas as pl
from jax.experimental.pallas import tpu as pltpu
from jax.experimental.pallas import tpu_sc as plsc

B = 16384
D = 16
V = 1000000
G = 8
CHUNK = 128
N_CHUNKS = 8


def _gather_body(uidx_hbm, iidx_hbm, utab_hbm, itab_hbm,
                 uout_hbm, iout_hbm,
                 idx_v, gidx_v, grp0_v, grp1_v, uout_v, iout_v,
                 sem0, sem1):
    wid = lax.axis_index("s") * 2 + lax.axis_index("c")

    pltpu.sync_copy(uidx_hbm.at[pl.ds(wid * 4, 4)], idx_v.at[pl.ds(0, 4)])
    pltpu.sync_copy(iidx_hbm.at[pl.ds(wid * 4, 4)], idx_v.at[pl.ds(4, 4)])

    for c in range(N_CHUNKS):
        for s in range(CHUNK // 16):
            gidx_v[c, pl.ds(s * 16, 16)] = (
                lax.shift_right_logical(idx_v[c, pl.ds(s * 16, 16)], 3))

    grps = (grp0_v, grp1_v)
    sems = (sem0, sem1)

    def fire(c):
        tab = utab_hbm if c < 4 else itab_hbm
        return pltpu.async_copy(tab.at[gidx_v.at[c]], grps[c % 2], sems[c % 2])

    def extract(c):
        grp = grps[c % 2]
        out = uout_v if c < 4 else iout_v
        cc = c % 4
        for s in range(CHUNK // 16):
            lv = lax.iota(jnp.int32, 16) + (s * 16)
            iv = idx_v[c, pl.ds(s * 16, 16)]
            colbase = (iv & 7) * 16
            orow = cc * 16 + lax.shift_right_logical(lv, 3)
            ocolbase = (lv & 7) * 16

            def kbody(k, _):
                v = plsc.load_gather(grp, [lv, colbase + k])
                plsc.store_scatter(out, [orow, ocolbase + k], v)
                return 0

            lax.fori_loop(0, 16, kbody, 0)

    pending = fire(0)
    for c in range(N_CHUNKS):
        pending.wait()
        if c + 1 < N_CHUNKS:
            pending = fire(c + 1)
        extract(c)

    pltpu.sync_copy(uout_v, uout_hbm.at[pl.ds(wid * 64, 64)])
    pltpu.sync_copy(iout_v, iout_hbm.at[pl.ds(wid * 64, 64)])


def _sc_gather(uidx2d, iidx2d, utab2, itab2):
    mesh = plsc.VectorSubcoreMesh(core_axis_name="c", subcore_axis_name="s")
    f = pl.kernel(
        _gather_body,
        mesh=mesh,
        compiler_params=pltpu.CompilerParams(needs_layout_passes=False),
        out_type=[
            jax.ShapeDtypeStruct((B // G, 128), jnp.float32),
            jax.ShapeDtypeStruct((B // G, 128), jnp.float32),
        ],
        scratch_types=[
            pltpu.VMEM((N_CHUNKS, CHUNK), jnp.int32),
            pltpu.VMEM((N_CHUNKS, CHUNK), jnp.int32),
            pltpu.VMEM((CHUNK, 128), jnp.float32),
            pltpu.VMEM((CHUNK, 128), jnp.float32),
            pltpu.VMEM((64, 128), jnp.float32),
            pltpu.VMEM((64, 128), jnp.float32),
            pltpu.SemaphoreType.DMA,
            pltpu.SemaphoreType.DMA,
        ],
    )
    return f(uidx2d, iidx2d, utab2, itab2)


def _mlp_body(u_ref, i_ref, w1u_ref, w1i_ref, b1_ref, w2t_ref, b2_ref,
              w3_ref, b3_ref, out_ref):
    x = (jnp.dot(u_ref[...], w1u_ref[...], preferred_element_type=jnp.float32)
         + jnp.dot(i_ref[...], w1i_ref[...], preferred_element_type=jnp.float32)
         + b1_ref[...])
    h1 = jnp.maximum(x, 0.0)
    h2 = jnp.maximum(
        jnp.dot(h1, w2t_ref[...], preferred_element_type=jnp.float32)
        + b2_ref[...], 0.0)
    out_ref[...] = jnp.sum(h2 * w3_ref[...], axis=1, keepdims=True) + b3_ref[...]


def _tc_mlp(u_emb, i_emb, w1u_t, w1i_t, b1, w2t, b2, w3, b3):
    blk = 2048
    grid = (B // blk,)
    full = lambda g: (0, 0)
    return pl.pallas_call(
        _mlp_body,
        grid=grid,
        in_specs=[
            pl.BlockSpec((blk, D), lambda g: (g, 0)),
            pl.BlockSpec((blk, D), lambda g: (g, 0)),
            pl.BlockSpec((D, 64), full),
            pl.BlockSpec((D, 64), full),
            pl.BlockSpec((1, 64), full),
            pl.BlockSpec((64, 32), full),
            pl.BlockSpec((1, 32), full),
            pl.BlockSpec((1, 32), full),
            pl.BlockSpec((1, 1), full),
        ],
        out_specs=pl.BlockSpec((blk, 1), lambda g: (g, 0)),
        out_shape=jax.ShapeDtypeStruct((B, 1), jnp.float32),
    )(u_emb, i_emb, w1u_t, w1i_t, b1, w2t, b2, w3, b3)


def kernel(user, item, user_table, item_table, W1, b1, W2, b2, W3, b3):
    uidx2d = user.astype(jnp.int32).reshape(B // CHUNK, CHUNK)
    iidx2d = item.astype(jnp.int32).reshape(B // CHUNK, CHUNK)
    utab2 = user_table.reshape(V // G, G * D)
    itab2 = item_table.reshape(V // G, G * D)
    u_pack, i_pack = _sc_gather(uidx2d, iidx2d, utab2, itab2)
    u_emb = u_pack.reshape(B, D)
    i_emb = i_pack.reshape(B, D)
    w1u_t = W1[:, :D].T
    w1i_t = W1[:, D:].T
    return _tc_mlp(u_emb, i_emb, w1u_t, w1i_t, b1.reshape(1, 64),
                   W2.T, b2.reshape(1, 32), W3, b3.reshape(1, 1))


# speedup vs baseline: 5.9112x; 5.9112x over previous
import jax
import jax.numpy as jnp
from jax import lax
from jax.experimental import pallas as pl
from jax.experimental.pallas import tpu as pltpu
from jax.experimental.pallas import tpu_sc as plsc

B = 16384
D = 16
V = 1000000
CHUNK = 128
W_IDX = 512


GRP = 32


def _gather_body(sidx_hbm, utab_hbm, itab_hbm,
                 uout_hbm, iout_hbm,
                 idx_v, grp_v, du_v, di_v, sem):
    wid = lax.axis_index("s") * 2 + lax.axis_index("c")

    pltpu.sync_copy(sidx_hbm.at[pl.ds(pl.multiple_of(wid * 8, 8), 8)], idx_v)

    def gather_tab(tab, dst, r0):
        def body(g, _):
            for s in range(GRP // 16):
                p = g * GRP + s * 16
                v = idx_v[r0 + lax.div(p, 128), pl.ds(lax.rem(p, 128), 16)]
                for j in range(16):
                    off = pl.multiple_of(
                        lax.shift_right_logical(v[j], 7) * 128, 128)
                    t = s * 16 + j
                    pltpu.make_async_copy(
                        tab.at[:, pl.ds(off, 128)],
                        grp_v.at[:, pl.ds(t * 128, 128)], sem).start()
            pltpu.make_async_copy(
                tab.at[:, pl.ds(0, GRP * 128)], grp_v, sem).wait()
            for s in range(GRP // 16):
                p = g * GRP + s * 16
                lv = idx_v[r0 + lax.div(p, 128),
                           pl.ds(lax.rem(p, 128), 16)] & 127
                colv = (lax.iota(jnp.int32, 16) + s * 16) * 128 + lv
                bv = lax.iota(jnp.int32, 16) + p

                def dbody(d, _):
                    dv = jnp.zeros((16,), jnp.int32) + d
                    vals = plsc.load_gather(grp_v, [dv, colv])
                    plsc.store_scatter(dst, [dv, bv], vals)
                    return 0

                lax.fori_loop(0, D, dbody, 0)
            return 0
        lax.fori_loop(0, W_IDX // GRP, body, 0)

    gather_tab(utab_hbm, du_v, 0)
    gather_tab(itab_hbm, di_v, 4)

    obase = pl.multiple_of(wid * W_IDX, 128)
    pltpu.sync_copy(du_v, uout_hbm.at[:, pl.ds(obase, W_IDX)])
    pltpu.sync_copy(di_v, iout_hbm.at[:, pl.ds(obase, W_IDX)])


def _sc_gather(sidx, utab_t, itab_t):
    mesh = plsc.VectorSubcoreMesh(core_axis_name="c", subcore_axis_name="s")
    f = pl.kernel(
        _gather_body,
        mesh=mesh,
        compiler_params=pltpu.CompilerParams(needs_layout_passes=False),
        out_type=[
            jax.ShapeDtypeStruct((D, B), jnp.float32),
            jax.ShapeDtypeStruct((D, B), jnp.float32),
        ],
        scratch_types=[
            pltpu.VMEM((8, CHUNK), jnp.int32),
            pltpu.VMEM((D, GRP * 128), jnp.float32),
            pltpu.VMEM((D, W_IDX), jnp.float32),
            pltpu.VMEM((D, W_IDX), jnp.float32),
            pltpu.SemaphoreType.DMA,
        ],
    )
    return f(sidx, utab_t, itab_t)


def _mlp_body(u_ref, i_ref, w1u_ref, w1i_ref, b1_ref, w2_ref, b2_ref,
              w3_ref, b3_ref, out_ref):
    x = (jnp.dot(w1u_ref[...], u_ref[...], preferred_element_type=jnp.float32)
         + jnp.dot(w1i_ref[...], i_ref[...], preferred_element_type=jnp.float32)
         + b1_ref[...])
    h1 = jnp.maximum(x, 0.0)
    h2 = jnp.maximum(
        jnp.dot(w2_ref[...], h1, preferred_element_type=jnp.float32)
        + b2_ref[...], 0.0)
    out_ref[...] = (
        jnp.dot(w3_ref[...], h2, preferred_element_type=jnp.float32)
        + b3_ref[...])


def _tc_mlp(u_t, i_t, w1u, w1i, b1, w2, b2, w3, b3):
    blk = 2048
    grid = (B // blk,)
    full = lambda g: (0, 0)
    return pl.pallas_call(
        _mlp_body,
        grid=grid,
        in_specs=[
            pl.BlockSpec((D, blk), lambda g: (0, g)),
            pl.BlockSpec((D, blk), lambda g: (0, g)),
            pl.BlockSpec((64, D), full),
            pl.BlockSpec((64, D), full),
            pl.BlockSpec((64, 1), full),
            pl.BlockSpec((32, 64), full),
            pl.BlockSpec((32, 1), full),
            pl.BlockSpec((1, 32), full),
            pl.BlockSpec((1, 1), full),
        ],
        out_specs=pl.BlockSpec((1, blk), lambda g: (0, g)),
        out_shape=jax.ShapeDtypeStruct((1, B), jnp.float32),
    )(u_t, i_t, w1u, w1i, b1, w2, b2, w3, b3)


def kernel(user, item, user_table, item_table, W1, b1, W2, b2, W3, b3):
    uidx = user.astype(jnp.int32).reshape(32, 4, CHUNK)
    iidx = item.astype(jnp.int32).reshape(32, 4, CHUNK)
    sidx = jnp.concatenate([uidx, iidx], axis=1).reshape(256, CHUNK)
    u_t, i_t = _sc_gather(sidx, user_table.T, item_table.T)
    out_t = _tc_mlp(u_t, i_t, W1[:, :D], W1[:, D:], b1.reshape(64, 1),
                    W2, b2.reshape(32, 1), W3, b3.reshape(1, 1))
    return out_t.reshape(B, 1)


# speedup vs baseline: 6.0780x; 1.0282x over previous
import jax
import jax.numpy as jnp
from jax import lax
from jax.experimental import pallas as pl
from jax.experimental.pallas import tpu as pltpu
from jax.experimental.pallas import tpu_sc as plsc

B = 16384
D = 16
V = 1000000
CHUNK = 128
W_IDX = 512


GRP = 16


def _gather_body(sidx_hbm, utab_hbm, itab_hbm,
                 uout_hbm, iout_hbm,
                 idx_v, grp0_v, grp1_v, du_v, di_v, sem0, sem1):
    wid = lax.axis_index("s") * 2 + lax.axis_index("c")

    pltpu.sync_copy(sidx_hbm.at[pl.ds(pl.multiple_of(wid * 8, 8), 8)], idx_v)

    grps = (grp0_v, grp1_v)
    sems = (sem0, sem1)
    n_grp = W_IDX // GRP

    def gather_tab(tab, dst, r0):
        def fire(g, buf):
            p = g * GRP
            v = idx_v[r0 + lax.div(p, 128), pl.ds(lax.rem(p, 128), 16)]
            for j in range(GRP):
                off = pl.multiple_of(
                    lax.shift_right_logical(v[j], 7) * 128, 128)
                pltpu.make_async_copy(
                    tab.at[:, pl.ds(off, 128)],
                    grps[buf].at[:, pl.ds(j * 128, 128)], sems[buf]).start()

        def drain_extract(g, buf):
            pltpu.make_async_copy(
                tab.at[:, pl.ds(0, GRP * 128)], grps[buf], sems[buf]).wait()
            p = g * GRP
            lv = idx_v[r0 + lax.div(p, 128), pl.ds(lax.rem(p, 128), 16)] & 127
            colv = lax.iota(jnp.int32, 16) * 128 + lv
            bv = lax.iota(jnp.int32, 16) + p

            def dbody(d, _):
                dv = jnp.zeros((16,), jnp.int32) + d
                vals = plsc.load_gather(grps[buf], [dv, colv])
                plsc.store_scatter(dst, [dv, bv], vals)
                return 0

            lax.fori_loop(0, D, dbody, 0)

        fire(0, 0)

        def body(h, _):
            g = h * 2
            fire(g + 1, 1)
            drain_extract(g, 0)
            fire(g + 2, 0)
            drain_extract(g + 1, 1)
            return 0

        lax.fori_loop(0, n_grp // 2 - 1, body, 0)
        g_last = n_grp - 2
        fire(g_last + 1, 1)
        drain_extract(g_last, 0)
        drain_extract(g_last + 1, 1)

    gather_tab(utab_hbm, du_v, 0)
    gather_tab(itab_hbm, di_v, 4)

    obase = pl.multiple_of(wid * W_IDX, 128)
    pltpu.sync_copy(du_v, uout_hbm.at[:, pl.ds(obase, W_IDX)])
    pltpu.sync_copy(di_v, iout_hbm.at[:, pl.ds(obase, W_IDX)])


def _sc_gather(sidx, utab_t, itab_t):
    mesh = plsc.VectorSubcoreMesh(core_axis_name="c", subcore_axis_name="s")
    f = pl.kernel(
        _gather_body,
        mesh=mesh,
        compiler_params=pltpu.CompilerParams(needs_layout_passes=False),
        out_type=[
            jax.ShapeDtypeStruct((D, B), jnp.float32),
            jax.ShapeDtypeStruct((D, B), jnp.float32),
        ],
        scratch_types=[
            pltpu.VMEM((8, CHUNK), jnp.int32),
            pltpu.VMEM((D, GRP * 128), jnp.float32),
            pltpu.VMEM((D, GRP * 128), jnp.float32),
            pltpu.VMEM((D, W_IDX), jnp.float32),
            pltpu.VMEM((D, W_IDX), jnp.float32),
            pltpu.SemaphoreType.DMA,
            pltpu.SemaphoreType.DMA,
        ],
    )
    return f(sidx, utab_t, itab_t)


def _mlp_body(u_ref, i_ref, w1u_ref, w1i_ref, b1_ref, w2_ref, b2_ref,
              w3_ref, b3_ref, out_ref):
    x = (jnp.dot(w1u_ref[...], u_ref[...], preferred_element_type=jnp.float32)
         + jnp.dot(w1i_ref[...], i_ref[...], preferred_element_type=jnp.float32)
         + b1_ref[...])
    h1 = jnp.maximum(x, 0.0)
    h2 = jnp.maximum(
        jnp.dot(w2_ref[...], h1, preferred_element_type=jnp.float32)
        + b2_ref[...], 0.0)
    out_ref[...] = (
        jnp.dot(w3_ref[...], h2, preferred_element_type=jnp.float32)
        + b3_ref[...])


def _tc_mlp(u_t, i_t, w1u, w1i, b1, w2, b2, w3, b3):
    return pl.pallas_call(
        _mlp_body,
        out_shape=jax.ShapeDtypeStruct((1, B), jnp.float32),
    )(u_t, i_t, w1u, w1i, b1, w2, b2, w3, b3)


def kernel(user, item, user_table, item_table, W1, b1, W2, b2, W3, b3):
    uidx = user.astype(jnp.int32).reshape(32, 4, CHUNK)
    iidx = item.astype(jnp.int32).reshape(32, 4, CHUNK)
    sidx = jnp.concatenate([uidx, iidx], axis=1).reshape(256, CHUNK)
    u_t, i_t = _sc_gather(sidx, user_table.T, item_table.T)
    out_t = _tc_mlp(u_t, i_t, W1[:, :D], W1[:, D:], b1.reshape(64, 1),
                    W2, b2.reshape(32, 1), W3, b3.reshape(1, 1))
    return out_t.reshape(B, 1)


# speedup vs baseline: 6.1553x; 1.0127x over previous
import jax
import jax.numpy as jnp
from jax import lax
from jax.experimental import pallas as pl
from jax.experimental.pallas import tpu as pltpu
from jax.experimental.pallas import tpu_sc as plsc

B = 16384
D = 16
V = 1000000
CHUNK = 128
W_IDX = 512


GRP = 16


def _gather_body(sidx_hbm, utab_hbm, itab_hbm,
                 uout_hbm, iout_hbm,
                 idx_v, grp0_v, grp1_v, du_v, di_v, sem0, sem1):
    wid = lax.axis_index("s") * 2 + lax.axis_index("c")

    pltpu.sync_copy(sidx_hbm.at[pl.ds(pl.multiple_of(wid * 8, 8), 8)], idx_v)

    grps = (grp0_v, grp1_v)
    sems = (sem0, sem1)
    n_grp = W_IDX // GRP

    def gather_tab(tab, dst, r0):
        def fire(g, buf):
            p = g * GRP
            v = idx_v[r0 + lax.div(p, 128), pl.ds(lax.rem(p, 128), 16)]
            for j in range(GRP):
                off = pl.multiple_of(
                    lax.shift_right_logical(v[j], 7) * 128, 128)
                pltpu.make_async_copy(
                    tab.at[pl.ds(0, 8), pl.ds(off, 128)],
                    grps[buf].at[pl.ds(0, 8), pl.ds(j * 128, 128)],
                    sems[buf]).start()
                pltpu.make_async_copy(
                    tab.at[pl.ds(8, 8), pl.ds(off, 128)],
                    grps[buf].at[pl.ds(8, 8), pl.ds(j * 128, 128)],
                    sems[buf]).start()

        def drain_extract(g, buf):
            pltpu.make_async_copy(
                tab.at[:, pl.ds(0, GRP * 128)], grps[buf], sems[buf]).wait()
            p = g * GRP
            lv = idx_v[r0 + lax.div(p, 128), pl.ds(lax.rem(p, 128), 16)] & 127
            colv = lax.iota(jnp.int32, 16) * 128 + lv
            bv = lax.iota(jnp.int32, 16) + p

            def dbody(d, _):
                dv = jnp.zeros((16,), jnp.int32) + d
                vals = plsc.load_gather(grps[buf], [dv, colv])
                plsc.store_scatter(dst, [dv, bv], vals)
                return 0

            lax.fori_loop(0, D, dbody, 0)

        fire(0, 0)

        def body(h, _):
            g = h * 2
            fire(g + 1, 1)
            drain_extract(g, 0)
            fire(g + 2, 0)
            drain_extract(g + 1, 1)
            return 0

        lax.fori_loop(0, n_grp // 2 - 1, body, 0)
        g_last = n_grp - 2
        fire(g_last + 1, 1)
        drain_extract(g_last, 0)
        drain_extract(g_last + 1, 1)

    gather_tab(utab_hbm, du_v, 0)
    gather_tab(itab_hbm, di_v, 4)

    obase = pl.multiple_of(wid * W_IDX, 128)
    pltpu.sync_copy(du_v, uout_hbm.at[:, pl.ds(obase, W_IDX)])
    pltpu.sync_copy(di_v, iout_hbm.at[:, pl.ds(obase, W_IDX)])


def _sc_gather(sidx, utab_t, itab_t):
    mesh = plsc.VectorSubcoreMesh(core_axis_name="c", subcore_axis_name="s")
    f = pl.kernel(
        _gather_body,
        mesh=mesh,
        compiler_params=pltpu.CompilerParams(needs_layout_passes=False),
        out_type=[
            jax.ShapeDtypeStruct((D, B), jnp.float32),
            jax.ShapeDtypeStruct((D, B), jnp.float32),
        ],
        scratch_types=[
            pltpu.VMEM((8, CHUNK), jnp.int32),
            pltpu.VMEM((D, GRP * 128), jnp.float32),
            pltpu.VMEM((D, GRP * 128), jnp.float32),
            pltpu.VMEM((D, W_IDX), jnp.float32),
            pltpu.VMEM((D, W_IDX), jnp.float32),
            pltpu.SemaphoreType.DMA,
            pltpu.SemaphoreType.DMA,
        ],
    )
    return f(sidx, utab_t, itab_t)


def _mlp_body(u_ref, i_ref, w1u_ref, w1i_ref, b1_ref, w2_ref, b2_ref,
              w3_ref, b3_ref, out_ref):
    x = (jnp.dot(w1u_ref[...], u_ref[...], preferred_element_type=jnp.float32)
         + jnp.dot(w1i_ref[...], i_ref[...], preferred_element_type=jnp.float32)
         + b1_ref[...])
    h1 = jnp.maximum(x, 0.0)
    h2 = jnp.maximum(
        jnp.dot(w2_ref[...], h1, preferred_element_type=jnp.float32)
        + b2_ref[...], 0.0)
    out_ref[...] = (
        jnp.dot(w3_ref[...], h2, preferred_element_type=jnp.float32)
        + b3_ref[...])


def _tc_mlp(u_t, i_t, w1u, w1i, b1, w2, b2, w3, b3):
    return pl.pallas_call(
        _mlp_body,
        out_shape=jax.ShapeDtypeStruct((1, B), jnp.float32),
    )(u_t, i_t, w1u, w1i, b1, w2, b2, w3, b3)


def kernel(user, item, user_table, item_table, W1, b1, W2, b2, W3, b3):
    uidx = user.astype(jnp.int32).reshape(32, 4, CHUNK)
    iidx = item.astype(jnp.int32).reshape(32, 4, CHUNK)
    sidx = jnp.concatenate([uidx, iidx], axis=1).reshape(256, CHUNK)
    u_t, i_t = _sc_gather(sidx, user_table.T, item_table.T)
    out_t = _tc_mlp(u_t, i_t, W1[:, :D], W1[:, D:], b1.reshape(64, 1),
                    W2, b2.reshape(32, 1), W3, b3.reshape(1, 1))
    return out_t.reshape(B, 1)
